# baseline (speedup 1.0000x reference)
import functools

import jax
import jax.numpy as jnp
from jax import lax
from jax.experimental import pallas as pl
from jax.experimental.pallas import tpu as pltpu
from jax.experimental.pallas import tpu_sc as plsc

N = 10000
E = 320000
D = 128

NC = 2
NS = 16
NW = NC * NS

CHUNK = 128
EDGES_PER_TILE = E // NW
FULL_CHUNKS = EDGES_PER_TILE // CHUNK
TAIL = EDGES_PER_TILE - FULL_CHUNKS * CHUNK
R = 2
S = 2 * R
NSEM = 2 * R + S + 4
ZTILES = 10
ZROWS = N // ZTILES

FULL_PAIRS = (FULL_CHUNKS - S - R) // S


def _sc_aggregate(src_hbm, dst_hbm, x_hbm, z_hbm, out_hbm,
                  acc, sidx, didx, rows, tsidx, tdidx, trows, sems):
    gsems = [sems.at[i] for i in range(R)]
    ssems = [sems.at[R + i] for i in range(R)]
    isems = [sems.at[2 * R + i] for i in range(S)]
    tis = sems.at[2 * R + S]
    tgs = sems.at[2 * R + S + 1]
    tss = sems.at[2 * R + S + 2]
    zsem = sems.at[2 * R + S + 3]
    cid = lax.axis_index("c")
    sid = lax.axis_index("s")
    wid = cid * NS + sid
    ebase = wid * EDGES_PER_TILE

    def idx_start(j, s):
        off = pl.multiple_of(ebase + j * CHUNK, 8)
        pltpu.async_copy(src_hbm.at[pl.ds(off, CHUNK)], sidx.at[s], isems[s])
        pltpu.async_copy(dst_hbm.at[pl.ds(off, CHUNK)], didx.at[s], isems[s])

    def idx_wait(s):
        pltpu.make_async_copy(src_hbm.at[pl.ds(0, CHUNK)], sidx.at[s],
                              isems[s]).wait()
        pltpu.make_async_copy(dst_hbm.at[pl.ds(0, CHUNK)], didx.at[s],
                              isems[s]).wait()

    def gather_start(b, s):
        pltpu.async_copy(x_hbm.at[sidx.at[s]], rows.at[b], gsems[b])

    def gather_wait(b):
        pltpu.make_async_copy(x_hbm.at[sidx.at[0]], rows.at[b],
                              gsems[b]).wait()

    def scatter_start(b, s):
        pltpu.async_copy(rows.at[b], acc.at[didx.at[s]], ssems[b], add=True)

    def scatter_wait(b):
        pltpu.make_async_copy(rows.at[b], acc.at[didx.at[0]],
                              ssems[b]).wait()

    def rblock(jb):
        for r in range(R):
            j = jb + r
            if j < FULL_CHUNKS:
                gather_wait(j % R)
                scatter_start(j % R, j % S)
        for r in range(R):
            j = jb + r
            if j < FULL_CHUNKS:
                scatter_wait(j % R)
                if j + S < FULL_CHUNKS:
                    idx_start(j + S, (j + S) % S)
                if j + R < FULL_CHUNKS:
                    idx_wait((j + R) % S)
                    gather_start((j + R) % R, (j + R) % S)

    def group(g, carry):
        t = g * S
        for k in range(S):
            if k % R == 0:
                for r in range(R):
                    gather_wait((k + r) % R)
                    scatter_start((k + r) % R, (k + r) % S)
            if k % R == R - 1:
                for r in range(R):
                    scatter_wait((k - R + 1 + r) % R)
                    idx_start(t + k - R + 1 + r + S, (k - R + 1 + r) % S)
                    idx_wait((k + 1 + r) % S)
                    gather_start((k + 1 + r) % R, (k + 1 + r) % S)
        return carry

    for s in range(S):
        idx_start(s, s)

    @pl.when(sid < ZTILES)
    def _zero():
        pltpu.async_copy(z_hbm, acc.at[pl.ds(sid * ZROWS, ZROWS)], zsem)

    for r in range(R):
        idx_wait(r)
        gather_start(r, r)

    @pl.when(sid < ZTILES)
    def _zero_wait():
        pltpu.make_async_copy(z_hbm, acc.at[pl.ds(0, ZROWS)], zsem).wait()

    plsc.subcore_barrier()

    lax.fori_loop(0, FULL_PAIRS, group, 0)

    for jb in range(FULL_PAIRS * S, FULL_CHUNKS + R - 1, R):
        rblock(jb)

    toff = pl.multiple_of(ebase + FULL_CHUNKS * CHUNK, 8)
    pltpu.async_copy(src_hbm.at[pl.ds(toff, TAIL)], tsidx, tis)
    pltpu.async_copy(dst_hbm.at[pl.ds(toff, TAIL)], tdidx, tis)
    pltpu.make_async_copy(src_hbm.at[pl.ds(0, TAIL)], tsidx, tis).wait()
    pltpu.make_async_copy(dst_hbm.at[pl.ds(0, TAIL)], tdidx, tis).wait()
    pltpu.async_copy(x_hbm.at[tsidx], trows, tgs).wait()
    pltpu.async_copy(trows, acc.at[tdidx], tss, add=True)
    pltpu.make_async_copy(trows, acc.at[tdidx], tss).wait()

    plsc.subcore_barrier()

    @pl.when(sid < ZTILES)
    def _writeout():
        row0 = sid * ZROWS
        out_off = pl.multiple_of(cid * N + row0, 8)
        pltpu.sync_copy(acc.at[pl.ds(row0, ZROWS)],
                        out_hbm.at[pl.ds(out_off, ZROWS)])


_sc_call = functools.partial(
    pl.kernel,
    out_type=jax.ShapeDtypeStruct((NC * N, D), jnp.float32),
    mesh=plsc.VectorSubcoreMesh(core_axis_name="c", subcore_axis_name="s",
                                num_cores=NC, num_subcores=NS),
    scratch_types=[
        pltpu.VMEM_SHARED((N, D), jnp.float32),
        pltpu.VMEM((S, CHUNK), jnp.int32),
        pltpu.VMEM((S, CHUNK), jnp.int32),
        pltpu.VMEM((R, CHUNK, D), jnp.float32),
        pltpu.VMEM((TAIL,), jnp.int32),
        pltpu.VMEM((TAIL,), jnp.int32),
        pltpu.VMEM((TAIL, D), jnp.float32),
        pltpu.SemaphoreType.DMA((NSEM,)),
    ],
)(_sc_aggregate)


def _tc_mlp(eps_ref, x_ref, p_ref, w1_ref, b1_ref, w2_ref, b2_ref, o_ref):
    h = x_ref[...] * (1.0 + eps_ref[0]) + p_ref[0] + p_ref[1]
    h = jnp.dot(h, w1_ref[...], preferred_element_type=jnp.float32)
    h = jnp.maximum(h + b1_ref[...], 0.0)
    h = jnp.dot(h, w2_ref[...], preferred_element_type=jnp.float32)
    h = jnp.maximum(h + b2_ref[...], 0.0)
    m = jnp.max(h, axis=-1, keepdims=True)
    lse = jnp.log(jnp.sum(jnp.exp(h - m), axis=-1, keepdims=True)) + m
    o_ref[...] = h - lse


BLK = 2000


def kernel(x, edge_index, eps, W1, b1, W2, b2):
    src = edge_index[0]
    dst = edge_index[1]
    zrows = jnp.zeros((ZROWS, D), jnp.float32)

    partials = _sc_call(src, dst, x, zrows)
    partials = partials.reshape(NC, N, D)

    grid = N // BLK
    out = pl.pallas_call(
        _tc_mlp,
        grid=(grid,),
        in_specs=[
            pl.BlockSpec(memory_space=pltpu.SMEM),
            pl.BlockSpec((BLK, D), lambda i: (i, 0)),
            pl.BlockSpec((NC, BLK, D), lambda i: (0, i, 0)),
            pl.BlockSpec((D, D), lambda i: (0, 0)),
            pl.BlockSpec((1, D), lambda i: (0, 0)),
            pl.BlockSpec((D, D), lambda i: (0, 0)),
            pl.BlockSpec((1, D), lambda i: (0, 0)),
        ],
        out_specs=pl.BlockSpec((BLK, D), lambda i: (i, 0)),
        out_shape=jax.ShapeDtypeStruct((N, D), jnp.float32),
    )(eps.reshape(1), x, partials, W1, b1.reshape(1, D), W2,
      b2.reshape(1, D))
    return out


# speedup vs baseline: 1.1938x; 1.1938x over previous
import functools

import jax
import jax.numpy as jnp
from jax import lax
from jax.experimental import pallas as pl
from jax.experimental.pallas import tpu as pltpu
from jax.experimental.pallas import tpu_sc as plsc

N = 10000
E = 320000
D = 128

NC = 2
NS = 16
NW = NC * NS

CHUNK = 64
EDGES_PER_TILE = E // NW
FULL_CHUNKS = EDGES_PER_TILE // CHUNK
TAIL = EDGES_PER_TILE - FULL_CHUNKS * CHUNK
R = 4
S = 2 * R
NSEM = 2 * R + S + 4
ZTILES = 10
ZROWS = N // ZTILES

FULL_PAIRS = (FULL_CHUNKS - S - R) // S


def _sc_aggregate(src_hbm, dst_hbm, x_hbm, z_hbm, out_hbm,
                  acc, sidx, didx, rows, tsidx, tdidx, trows, sems):
    gsems = [sems.at[i] for i in range(R)]
    ssems = [sems.at[R + i] for i in range(R)]
    isems = [sems.at[2 * R + i] for i in range(S)]
    tis = sems.at[2 * R + S]
    tgs = sems.at[2 * R + S + 1]
    tss = sems.at[2 * R + S + 2]
    zsem = sems.at[2 * R + S + 3]
    cid = lax.axis_index("c")
    sid = lax.axis_index("s")
    wid = cid * NS + sid
    ebase = wid * EDGES_PER_TILE

    def idx_start(j, s):
        off = pl.multiple_of(ebase + j * CHUNK, 8)
        pltpu.async_copy(src_hbm.at[pl.ds(off, CHUNK)], sidx.at[s], isems[s])
        pltpu.async_copy(dst_hbm.at[pl.ds(off, CHUNK)], didx.at[s], isems[s])

    def idx_wait(s):
        pltpu.make_async_copy(src_hbm.at[pl.ds(0, CHUNK)], sidx.at[s],
                              isems[s]).wait()
        pltpu.make_async_copy(dst_hbm.at[pl.ds(0, CHUNK)], didx.at[s],
                              isems[s]).wait()

    def gather_start(b, s):
        pltpu.async_copy(x_hbm.at[sidx.at[s]], rows.at[b], gsems[b])

    def gather_wait(b):
        pltpu.make_async_copy(x_hbm.at[sidx.at[0]], rows.at[b],
                              gsems[b]).wait()

    def scatter_start(b, s):
        pltpu.async_copy(rows.at[b], acc.at[didx.at[s]], ssems[b], add=True)

    def scatter_wait(b):
        pltpu.make_async_copy(rows.at[b], acc.at[didx.at[0]],
                              ssems[b]).wait()

    def rblock(jb):
        for r in range(R):
            j = jb + r
            if j < FULL_CHUNKS:
                gather_wait(j % R)
                scatter_start(j % R, j % S)
        for r in range(R):
            j = jb + r
            if j < FULL_CHUNKS:
                scatter_wait(j % R)
                if j + S < FULL_CHUNKS:
                    idx_start(j + S, (j + S) % S)
                if j + R < FULL_CHUNKS:
                    idx_wait((j + R) % S)
                    gather_start((j + R) % R, (j + R) % S)

    def group(g, carry):
        t = g * S
        for k in range(S):
            if k % R == 0:
                for r in range(R):
                    gather_wait((k + r) % R)
                    scatter_start((k + r) % R, (k + r) % S)
            if k % R == R - 1:
                for r in range(R):
                    scatter_wait((k - R + 1 + r) % R)
                    idx_start(t + k - R + 1 + r + S, (k - R + 1 + r) % S)
                    idx_wait((k + 1 + r) % S)
                    gather_start((k + 1 + r) % R, (k + 1 + r) % S)
        return carry

    for s in range(S):
        idx_start(s, s)

    @pl.when(sid < ZTILES)
    def _zero():
        pltpu.async_copy(z_hbm, acc.at[pl.ds(sid * ZROWS, ZROWS)], zsem)

    for r in range(R):
        idx_wait(r)
        gather_start(r, r)

    @pl.when(sid < ZTILES)
    def _zero_wait():
        pltpu.make_async_copy(z_hbm, acc.at[pl.ds(0, ZROWS)], zsem).wait()

    plsc.subcore_barrier()

    lax.fori_loop(0, FULL_PAIRS, group, 0)

    for jb in range(FULL_PAIRS * S, FULL_CHUNKS + R - 1, R):
        rblock(jb)

    toff = pl.multiple_of(ebase + FULL_CHUNKS * CHUNK, 8)
    pltpu.async_copy(src_hbm.at[pl.ds(toff, TAIL)], tsidx, tis)
    pltpu.async_copy(dst_hbm.at[pl.ds(toff, TAIL)], tdidx, tis)
    pltpu.make_async_copy(src_hbm.at[pl.ds(0, TAIL)], tsidx, tis).wait()
    pltpu.make_async_copy(dst_hbm.at[pl.ds(0, TAIL)], tdidx, tis).wait()
    pltpu.async_copy(x_hbm.at[tsidx], trows, tgs).wait()
    pltpu.async_copy(trows, acc.at[tdidx], tss, add=True)
    pltpu.make_async_copy(trows, acc.at[tdidx], tss).wait()

    plsc.subcore_barrier()

    @pl.when(sid < ZTILES)
    def _writeout():
        row0 = sid * ZROWS
        out_off = pl.multiple_of(cid * N + row0, 8)
        pltpu.sync_copy(acc.at[pl.ds(row0, ZROWS)],
                        out_hbm.at[pl.ds(out_off, ZROWS)])


_sc_call = functools.partial(
    pl.kernel,
    out_type=jax.ShapeDtypeStruct((NC * N, D), jnp.float32),
    mesh=plsc.VectorSubcoreMesh(core_axis_name="c", subcore_axis_name="s",
                                num_cores=NC, num_subcores=NS),
    scratch_types=[
        pltpu.VMEM_SHARED((N, D), jnp.float32),
        pltpu.VMEM((S, CHUNK), jnp.int32),
        pltpu.VMEM((S, CHUNK), jnp.int32),
        pltpu.VMEM((R, CHUNK, D), jnp.float32),
        pltpu.VMEM((TAIL,), jnp.int32),
        pltpu.VMEM((TAIL,), jnp.int32),
        pltpu.VMEM((TAIL, D), jnp.float32),
        pltpu.SemaphoreType.DMA((NSEM,)),
    ],
)(_sc_aggregate)


def _tc_mlp(eps_ref, x_ref, p_ref, w1_ref, b1_ref, w2_ref, b2_ref, o_ref):
    h = x_ref[...] * (1.0 + eps_ref[0]) + p_ref[0] + p_ref[1]
    h = jnp.dot(h, w1_ref[...], preferred_element_type=jnp.float32)
    h = jnp.maximum(h + b1_ref[...], 0.0)
    h = jnp.dot(h, w2_ref[...], preferred_element_type=jnp.float32)
    h = jnp.maximum(h + b2_ref[...], 0.0)
    m = jnp.max(h, axis=-1, keepdims=True)
    lse = jnp.log(jnp.sum(jnp.exp(h - m), axis=-1, keepdims=True)) + m
    o_ref[...] = h - lse


BLK = 2000


def kernel(x, edge_index, eps, W1, b1, W2, b2):
    src = edge_index[0]
    dst = edge_index[1]
    zrows = jnp.zeros((ZROWS, D), jnp.float32)

    partials = _sc_call(src, dst, x, zrows)
    partials = partials.reshape(NC, N, D)

    grid = N // BLK
    out = pl.pallas_call(
        _tc_mlp,
        grid=(grid,),
        in_specs=[
            pl.BlockSpec(memory_space=pltpu.SMEM),
            pl.BlockSpec((BLK, D), lambda i: (i, 0)),
            pl.BlockSpec((NC, BLK, D), lambda i: (0, i, 0)),
            pl.BlockSpec((D, D), lambda i: (0, 0)),
            pl.BlockSpec((1, D), lambda i: (0, 0)),
            pl.BlockSpec((D, D), lambda i: (0, 0)),
            pl.BlockSpec((1, D), lambda i: (0, 0)),
        ],
        out_specs=pl.BlockSpec((BLK, D), lambda i: (i, 0)),
        out_shape=jax.ShapeDtypeStruct((N, D), jnp.float32),
    )(eps.reshape(1), x, partials, W1, b1.reshape(1, D), W2,
      b2.reshape(1, D))
    return out


# speedup vs baseline: 1.2218x; 1.0234x over previous
import functools

import jax
import jax.numpy as jnp
from jax import lax
from jax.experimental import pallas as pl
from jax.experimental.pallas import tpu as pltpu
from jax.experimental.pallas import tpu_sc as plsc

N = 10000
E = 320000
D = 128

NC = 2
NS = 16
NW = NC * NS

CHUNK = 64
EDGES_PER_TILE = E // NW
FULL_CHUNKS = EDGES_PER_TILE // CHUNK
TAIL = EDGES_PER_TILE - FULL_CHUNKS * CHUNK
R = 5
S = 2 * R
NSEM = 2 * R + S + 4
ZTILES = 10
ZROWS = N // ZTILES

FULL_PAIRS = (FULL_CHUNKS - S - R) // S


def _sc_aggregate(src_hbm, dst_hbm, x_hbm, z_hbm, out_hbm,
                  acc, sidx, didx, rows, tsidx, tdidx, trows, sems):
    gsems = [sems.at[i] for i in range(R)]
    ssems = [sems.at[R + i] for i in range(R)]
    isems = [sems.at[2 * R + i] for i in range(S)]
    tis = sems.at[2 * R + S]
    tgs = sems.at[2 * R + S + 1]
    tss = sems.at[2 * R + S + 2]
    zsem = sems.at[2 * R + S + 3]
    cid = lax.axis_index("c")
    sid = lax.axis_index("s")
    wid = cid * NS + sid
    ebase = wid * EDGES_PER_TILE

    def idx_start(j, s):
        off = pl.multiple_of(ebase + j * CHUNK, 8)
        pltpu.async_copy(src_hbm.at[pl.ds(off, CHUNK)], sidx.at[s], isems[s])
        pltpu.async_copy(dst_hbm.at[pl.ds(off, CHUNK)], didx.at[s], isems[s])

    def idx_wait(s):
        pltpu.make_async_copy(src_hbm.at[pl.ds(0, CHUNK)], sidx.at[s],
                              isems[s]).wait()
        pltpu.make_async_copy(dst_hbm.at[pl.ds(0, CHUNK)], didx.at[s],
                              isems[s]).wait()

    def gather_start(b, s):
        pltpu.async_copy(x_hbm.at[sidx.at[s]], rows.at[b], gsems[b])

    def gather_wait(b):
        pltpu.make_async_copy(x_hbm.at[sidx.at[0]], rows.at[b],
                              gsems[b]).wait()

    def scatter_start(b, s):
        pltpu.async_copy(rows.at[b], acc.at[didx.at[s]], ssems[b], add=True)

    def scatter_wait(b):
        pltpu.make_async_copy(rows.at[b], acc.at[didx.at[0]],
                              ssems[b]).wait()

    def rblock(jb):
        for r in range(R):
            j = jb + r
            if j < FULL_CHUNKS:
                gather_wait(j % R)
                scatter_start(j % R, j % S)
        for r in range(R):
            j = jb + r
            if j < FULL_CHUNKS:
                scatter_wait(j % R)
                if j + S < FULL_CHUNKS:
                    idx_start(j + S, (j + S) % S)
                if j + R < FULL_CHUNKS:
                    idx_wait((j + R) % S)
                    gather_start((j + R) % R, (j + R) % S)

    def group(g, carry):
        t = g * S
        for k in range(S):
            if k % R == 0:
                for r in range(R):
                    gather_wait((k + r) % R)
                    scatter_start((k + r) % R, (k + r) % S)
            if k % R == R - 1:
                for r in range(R):
                    scatter_wait((k - R + 1 + r) % R)
                    idx_start(t + k - R + 1 + r + S, (k - R + 1 + r) % S)
                    idx_wait((k + 1 + r) % S)
                    gather_start((k + 1 + r) % R, (k + 1 + r) % S)
        return carry

    for s in range(S):
        idx_start(s, s)

    @pl.when(sid < ZTILES)
    def _zero():
        pltpu.async_copy(z_hbm, acc.at[pl.ds(sid * ZROWS, ZROWS)], zsem)

    for r in range(R):
        idx_wait(r)
        gather_start(r, r)

    @pl.when(sid < ZTILES)
    def _zero_wait():
        pltpu.make_async_copy(z_hbm, acc.at[pl.ds(0, ZROWS)], zsem).wait()

    plsc.subcore_barrier()

    lax.fori_loop(0, FULL_PAIRS, group, 0)

    for jb in range(FULL_PAIRS * S, FULL_CHUNKS + R - 1, R):
        rblock(jb)

    toff = pl.multiple_of(ebase + FULL_CHUNKS * CHUNK, 8)
    pltpu.async_copy(src_hbm.at[pl.ds(toff, TAIL)], tsidx, tis)
    pltpu.async_copy(dst_hbm.at[pl.ds(toff, TAIL)], tdidx, tis)
    pltpu.make_async_copy(src_hbm.at[pl.ds(0, TAIL)], tsidx, tis).wait()
    pltpu.make_async_copy(dst_hbm.at[pl.ds(0, TAIL)], tdidx, tis).wait()
    pltpu.async_copy(x_hbm.at[tsidx], trows, tgs).wait()
    pltpu.async_copy(trows, acc.at[tdidx], tss, add=True)
    pltpu.make_async_copy(trows, acc.at[tdidx], tss).wait()

    plsc.subcore_barrier()

    @pl.when(sid < ZTILES)
    def _writeout():
        row0 = sid * ZROWS
        out_off = pl.multiple_of(cid * N + row0, 8)
        pltpu.sync_copy(acc.at[pl.ds(row0, ZROWS)],
                        out_hbm.at[pl.ds(out_off, ZROWS)])


_sc_call = functools.partial(
    pl.kernel,
    out_type=jax.ShapeDtypeStruct((NC * N, D), jnp.float32),
    mesh=plsc.VectorSubcoreMesh(core_axis_name="c", subcore_axis_name="s",
                                num_cores=NC, num_subcores=NS),
    scratch_types=[
        pltpu.VMEM_SHARED((N, D), jnp.float32),
        pltpu.VMEM((S, CHUNK), jnp.int32),
        pltpu.VMEM((S, CHUNK), jnp.int32),
        pltpu.VMEM((R, CHUNK, D), jnp.float32),
        pltpu.VMEM((TAIL,), jnp.int32),
        pltpu.VMEM((TAIL,), jnp.int32),
        pltpu.VMEM((TAIL, D), jnp.float32),
        pltpu.SemaphoreType.DMA((NSEM,)),
    ],
)(_sc_aggregate)


def _tc_mlp(eps_ref, x_ref, p_ref, w1_ref, b1_ref, w2_ref, b2_ref, o_ref):
    h = x_ref[...] * (1.0 + eps_ref[0]) + p_ref[0] + p_ref[1]
    h = jnp.dot(h, w1_ref[...], preferred_element_type=jnp.float32)
    h = jnp.maximum(h + b1_ref[...], 0.0)
    h = jnp.dot(h, w2_ref[...], preferred_element_type=jnp.float32)
    h = jnp.maximum(h + b2_ref[...], 0.0)
    m = jnp.max(h, axis=-1, keepdims=True)
    lse = jnp.log(jnp.sum(jnp.exp(h - m), axis=-1, keepdims=True)) + m
    o_ref[...] = h - lse


BLK = 2000


def kernel(x, edge_index, eps, W1, b1, W2, b2):
    src = edge_index[0]
    dst = edge_index[1]
    zrows = jnp.zeros((ZROWS, D), jnp.float32)

    partials = _sc_call(src, dst, x, zrows)
    partials = partials.reshape(NC, N, D)

    grid = N // BLK
    out = pl.pallas_call(
        _tc_mlp,
        grid=(grid,),
        in_specs=[
            pl.BlockSpec(memory_space=pltpu.SMEM),
            pl.BlockSpec((BLK, D), lambda i: (i, 0)),
            pl.BlockSpec((NC, BLK, D), lambda i: (0, i, 0)),
            pl.BlockSpec((D, D), lambda i: (0, 0)),
            pl.BlockSpec((1, D), lambda i: (0, 0)),
            pl.BlockSpec((D, D), lambda i: (0, 0)),
            pl.BlockSpec((1, D), lambda i: (0, 0)),
        ],
        out_specs=pl.BlockSpec((BLK, D), lambda i: (i, 0)),
        out_shape=jax.ShapeDtypeStruct((N, D), jnp.float32),
    )(eps.reshape(1), x, partials, W1, b1.reshape(1, D), W2,
      b2.reshape(1, D))
    return out
